# baseline (speedup 1.0000x reference)
import functools

import jax
import jax.numpy as jnp
from jax import lax
from jax.experimental import pallas as pl
from jax.experimental.pallas import tpu as pltpu
from jax.experimental.pallas import tpu_sc as plsc

_K = 1024
_D = 32
_B = 4096
_BETA = 0.25
_EPS = 1e-08
_BT = 2048
_NBLK = _B // _BT

_NC = 2
_NS = 16
_NW = _NC * _NS
_BPW = _B // _NW

_DPAD = 128


def _argmin_body(lt_ref, pt_ref, idx_ref, ptab_ref):
    lt = lt_ref[...]
    pt = pt_ref[...]

    @pl.when(pl.program_id(0) == 0)
    def _build_table():
        ptab_ref[...] = jnp.concatenate(
            [pt.T, jnp.zeros((_K, _DPAD - _D), jnp.float32)], axis=1)

    pn = jnp.sum(pt * pt, axis=0).reshape(_K, 1)
    ln = jnp.sum(lt * lt, axis=0, keepdims=True)
    mm1 = lax.dot_general(pt, lt, (((0,), (0,)), ((), ())),
                          preferred_element_type=jnp.float32)
    d1 = (ln + pn) - 2.0 * mm1
    colmin = jnp.min(d1, axis=0, keepdims=True)
    rowid_f = lax.broadcasted_iota(jnp.int32, (_K, _BT), 0).astype(jnp.float32)
    idx_f = jnp.min(jnp.where(d1 == colmin, rowid_f, float(_K)), axis=0)
    idx_ref[...] = idx_f.astype(jnp.int32)


_argmin_call = pl.pallas_call(
    _argmin_body,
    grid=(_NBLK,),
    in_specs=[
        pl.BlockSpec((_D, _BT), lambda i: (0, i)),
        pl.BlockSpec((_D, _K), lambda i: (0, 0)),
    ],
    out_specs=[
        pl.BlockSpec((_BT,), lambda i: (i,)),
        pl.BlockSpec((_K, _DPAD), lambda i: (0, 0)),
    ],
    out_shape=[
        jax.ShapeDtypeStruct((_B,), jnp.int32),
        jax.ShapeDtypeStruct((_K, _DPAD), jnp.float32),
    ],
    compiler_params=pltpu.CompilerParams(
        dimension_semantics=("arbitrary",),
    ),
)


def _ent_body(mt_ref, pt_ref, ent_ref, pacc_ref):
    i = pl.program_id(0)

    @pl.when(i == 0)
    def _init():
        pacc_ref[...] = jnp.zeros_like(pacc_ref)

    mt = mt_ref[...]
    pt = pt_ref[...]

    pn = jnp.sum(pt * pt, axis=0, keepdims=True)
    mn = jnp.sum(mt * mt, axis=0, keepdims=True)
    onesk = jnp.ones((1, _K), dtype=jnp.float32)
    onesb = jnp.ones((1, _BT), dtype=jnp.float32)
    lhs = jnp.concatenate([pt * -2.0, pn, onesk], axis=0)
    rhs = jnp.concatenate([mt, onesb, mn], axis=0)
    d2 = lax.dot_general(lhs, rhs, (((0,), (0,)), ((), ())),
                         preferred_element_type=jnp.float32)

    e = jnp.exp(-d2)
    inv_rs = 1.0 / (jnp.sum(e, axis=0, keepdims=True) + (_K * _EPS))
    pacc_ref[...] += (jnp.sum(e * inv_rs, axis=1, keepdims=True)
                      + _EPS * jnp.sum(inv_rs))

    @pl.when(i == _NBLK - 1)
    def _finish():
        approx = pacc_ref[...] / _B
        ent = -jnp.sum(approx * jnp.log(approx))
        ent_ref[...] = jnp.full((1, 1), ent, dtype=jnp.float32)


_ent_call = pl.pallas_call(
    _ent_body,
    grid=(_NBLK,),
    in_specs=[
        pl.BlockSpec((_D, _BT), lambda i: (0, i)),
        pl.BlockSpec((_D, _K), lambda i: (0, 0)),
    ],
    out_specs=pl.BlockSpec((1, 1), lambda i: (0, 0)),
    out_shape=jax.ShapeDtypeStruct((1, 1), jnp.float32),
    scratch_shapes=[
        pltpu.VMEM((_K, 1), jnp.float32),
    ],
    compiler_params=pltpu.CompilerParams(
        dimension_semantics=("arbitrary",),
    ),
)


@functools.cache
def _make_sc_gather():
    @functools.partial(
        pl.kernel,
        out_type=[
            jax.ShapeDtypeStruct((_D, _B), jnp.float32),
            jax.ShapeDtypeStruct((_NW, 16), jnp.float32),
        ],
        mesh=plsc.VectorSubcoreMesh(core_axis_name="c", subcore_axis_name="s",
                                    num_cores=_NC, num_subcores=_NS),
        scratch_types=[
            pltpu.VMEM((_BPW,), jnp.int32),
            pltpu.VMEM((_BPW, _DPAD), jnp.float32),
            pltpu.VMEM((_D, _BPW), jnp.float32),
            pltpu.VMEM((16,), jnp.float32),
            pltpu.SemaphoreType.DMA,
        ],
        compiler_params=pltpu.CompilerParams(needs_layout_passes=False),
    )
    def _sc_gather(table_hbm, idx_hbm, mt_hbm, out_hbm, mse_hbm,
                   idx_v, rows_v, outt_v, acc_v, sem):
        wid = lax.axis_index("s") * _NC + lax.axis_index("c")
        base = wid * _BPW
        pltpu.sync_copy(idx_hbm.at[pl.ds(base, _BPW)], idx_v)
        pltpu.sync_copy(mt_hbm.at[:, pl.ds(base, _BPW)], outt_v)
        pltpu.async_copy(table_hbm.at[idx_v], rows_v, sem).wait()
        lanes = lax.iota(jnp.int32, 16)
        acc = jnp.zeros((16,), jnp.float32)
        for c in range(_BPW // 16):
            rows16 = lanes + (c * 16)
            for d in range(_D):
                q = plsc.load_gather(
                    rows_v, [rows16, jnp.full((16,), d, jnp.int32)])
                m = outt_v[d, pl.ds(c * 16, 16)]
                diff = q - m
                acc = acc + diff * diff
                outt_v[d, pl.ds(c * 16, 16)] = q
        acc_v[...] = acc
        pltpu.sync_copy(outt_v, out_hbm.at[:, pl.ds(base, _BPW)])
        pltpu.sync_copy(acc_v, mse_hbm.at[wid])

    return _sc_gather


def kernel(latents, mus, prototypes):
    lt = latents.T
    mt = mus.T
    pt = prototypes.T
    idx, table = _argmin_call(lt, pt)
    quantized_t, mse_parts = _make_sc_gather()(table, idx, mt)
    ent = _ent_call(mt, pt)
    vq_loss = (ent.reshape(())
               + (1.0 + _BETA) * (jnp.sum(mse_parts) / (_B * _D)))
    return quantized_t.T, vq_loss


# speedup vs baseline: 1.1007x; 1.1007x over previous
import functools

import jax
import jax.numpy as jnp
from jax import lax
from jax.experimental import pallas as pl
from jax.experimental.pallas import tpu as pltpu
from jax.experimental.pallas import tpu_sc as plsc

_K = 1024
_D = 32
_B = 4096
_BETA = 0.25
_EPS = 1e-08
_BT = 2048
_NBLK = _B // _BT

_NC = 2
_NS = 16
_NW = _NC * _NS
_BPW = _B // _NW

_DPAD = 128


def _argmin_body(lt_ref, pt_ref, idx_ref, ptab_ref):
    lt = lt_ref[...]
    pt = pt_ref[...]

    @pl.when(pl.program_id(0) == 0)
    def _build_table():
        ptab_ref[...] = jnp.concatenate(
            [pt.T, jnp.zeros((_K, _DPAD - _D), jnp.float32)], axis=1)

    pn = jnp.sum(pt * pt, axis=0).reshape(_K, 1)
    ln = jnp.sum(lt * lt, axis=0, keepdims=True)
    mm1 = lax.dot_general(pt, lt, (((0,), (0,)), ((), ())),
                          preferred_element_type=jnp.float32)
    d1 = (ln + pn) - 2.0 * mm1
    colmin = jnp.min(d1, axis=0, keepdims=True)
    rowid_f = lax.broadcasted_iota(jnp.int32, (_K, _BT), 0).astype(jnp.float32)
    idx_f = jnp.min(jnp.where(d1 == colmin, rowid_f, float(_K)), axis=0)
    idx_ref[...] = idx_f.astype(jnp.int32)


_argmin_call = pl.pallas_call(
    _argmin_body,
    grid=(_NBLK,),
    in_specs=[
        pl.BlockSpec((_D, _BT), lambda i: (0, i)),
        pl.BlockSpec((_D, _K), lambda i: (0, 0)),
    ],
    out_specs=[
        pl.BlockSpec((_BT,), lambda i: (i,)),
        pl.BlockSpec((_K, _DPAD), lambda i: (0, 0)),
    ],
    out_shape=[
        jax.ShapeDtypeStruct((_B,), jnp.int32),
        jax.ShapeDtypeStruct((_K, _DPAD), jnp.float32),
    ],
    compiler_params=pltpu.CompilerParams(
        dimension_semantics=("arbitrary",),
    ),
)


def _ent_body(mt_ref, pt_ref, idx_ref, ent_ref, pacc_ref, macc_ref):
    i = pl.program_id(0)

    @pl.when(i == 0)
    def _init():
        pacc_ref[...] = jnp.zeros_like(pacc_ref)
        macc_ref[0, 0] = 0.0

    mt = mt_ref[...]
    pt = pt_ref[...]
    idx = idx_ref[...].reshape(1, _BT)

    pn = jnp.sum(pt * pt, axis=0, keepdims=True)
    mn = jnp.sum(mt * mt, axis=0, keepdims=True)
    onesk = jnp.ones((1, _K), dtype=jnp.float32)
    onesb = jnp.ones((1, _BT), dtype=jnp.float32)
    lhs = jnp.concatenate([pt * -2.0, pn, onesk], axis=0)
    rhs = jnp.concatenate([mt, onesb, mn], axis=0)
    d2 = lax.dot_general(lhs, rhs, (((0,), (0,)), ((), ())),
                         preferred_element_type=jnp.float32)

    rowid = lax.broadcasted_iota(jnp.int32, (_K, _BT), 0)
    macc_ref[0, 0] += jnp.sum(jnp.where(rowid == idx, d2, 0.0))

    e = jnp.exp(-d2)
    inv_rs = 1.0 / (jnp.sum(e, axis=0, keepdims=True) + (_K * _EPS))
    pacc_ref[...] += (jnp.sum(e * inv_rs, axis=1, keepdims=True)
                      + _EPS * jnp.sum(inv_rs))

    @pl.when(i == _NBLK - 1)
    def _finish():
        approx = pacc_ref[...] / _B
        ent = -jnp.sum(approx * jnp.log(approx))
        mse_mean = macc_ref[0, 0] / (_B * _D)
        loss = (1.0 + _BETA) * mse_mean + ent
        ent_ref[...] = jnp.full((1, 1), loss, dtype=jnp.float32)


_ent_call = pl.pallas_call(
    _ent_body,
    grid=(_NBLK,),
    in_specs=[
        pl.BlockSpec((_D, _BT), lambda i: (0, i)),
        pl.BlockSpec((_D, _K), lambda i: (0, 0)),
        pl.BlockSpec((_BT,), lambda i: (i,)),
    ],
    out_specs=pl.BlockSpec((1, 1), lambda i: (0, 0)),
    out_shape=jax.ShapeDtypeStruct((1, 1), jnp.float32),
    scratch_shapes=[
        pltpu.VMEM((_K, 1), jnp.float32),
        pltpu.SMEM((1, 1), jnp.float32),
    ],
    compiler_params=pltpu.CompilerParams(
        dimension_semantics=("arbitrary",),
    ),
)


@functools.cache
def _make_sc_gather():
    @functools.partial(
        pl.kernel,
        out_type=jax.ShapeDtypeStruct((_D, _B), jnp.float32),
        mesh=plsc.VectorSubcoreMesh(core_axis_name="c", subcore_axis_name="s",
                                    num_cores=_NC, num_subcores=_NS),
        scratch_types=[
            pltpu.VMEM((_BPW,), jnp.int32),
            pltpu.VMEM((_BPW, _DPAD), jnp.float32),
            pltpu.VMEM((_D, _BPW), jnp.float32),
            pltpu.SemaphoreType.DMA,
        ],
        compiler_params=pltpu.CompilerParams(needs_layout_passes=False),
    )
    def _sc_gather(table_hbm, idx_hbm, out_hbm, idx_v, rows_v, outt_v, sem):
        wid = lax.axis_index("s") * _NC + lax.axis_index("c")
        base = wid * _BPW
        pltpu.sync_copy(idx_hbm.at[pl.ds(base, _BPW)], idx_v)
        pltpu.async_copy(table_hbm.at[idx_v], rows_v, sem).wait()
        lanes = lax.iota(jnp.int32, 16)
        for c in range(_BPW // 16):
            rows16 = lanes + (c * 16)
            for d in range(_D):
                q = plsc.load_gather(
                    rows_v, [rows16, jnp.full((16,), d, jnp.int32)])
                outt_v[d, pl.ds(c * 16, 16)] = q
        pltpu.sync_copy(outt_v, out_hbm.at[:, pl.ds(base, _BPW)])

    return _sc_gather


def kernel(latents, mus, prototypes):
    lt = latents.T
    mt = mus.T
    pt = prototypes.T
    idx, table = _argmin_call(lt, pt)
    quantized_t = _make_sc_gather()(table, idx)
    loss = _ent_call(mt, pt, idx)
    return quantized_t.T, loss.reshape(())
